# baseline (speedup 1.0000x reference)
import functools

import jax
import jax.numpy as jnp
from jax import lax
from jax.experimental import pallas as pl
from jax.experimental.pallas import tpu as pltpu
from jax.experimental.pallas import tpu_sc as plsc

DIM = 64
MARGIN = 9.0
NC = 2
NS = 16
L = 16
NW = NC * NS
CHUNK = 128


def _kge_sc(head_r, rel_r, tail_r, entity_emb, relation_emb, e_bias, r_bias,
            batch):
    bpw = batch // NW
    nchunks = bpw // CHUNK

    mesh = plsc.VectorSubcoreMesh(core_axis_name="c", subcore_axis_name="s")

    @functools.partial(
        pl.kernel,
        mesh=mesh,
        out_type=jax.ShapeDtypeStruct((batch,), jnp.float32),
        compiler_params=pltpu.CompilerParams(
            use_tc_tiling_on_sc=False,
            needs_layout_passes=False,
        ),
        scratch_types=[
            pltpu.VMEM((nchunks, CHUNK), jnp.int32),
            pltpu.VMEM((nchunks, CHUNK), jnp.int32),
            pltpu.VMEM((nchunks, CHUNK), jnp.int32),
            pltpu.VMEM((bpw, DIM), jnp.float32),
            pltpu.VMEM((bpw, DIM), jnp.float32),
            pltpu.VMEM((bpw, DIM), jnp.float32),
            pltpu.VMEM((bpw,), jnp.float32),
            pltpu.VMEM((bpw,), jnp.float32),
            pltpu.VMEM((bpw,), jnp.float32),
            pltpu.VMEM((bpw,), jnp.float32),
            pltpu.SemaphoreType.DMA,
            pltpu.SemaphoreType.DMA,
            pltpu.SemaphoreType.DMA,
            pltpu.SemaphoreType.DMA,
        ],
    )
    def kge(head_hbm, rel_hbm, tail_hbm, ent_hbm, relemb_hbm, eb_hbm, rb_hbm,
            out_hbm, hidx, ridx, tidx, hrow, rrow, trow, hb, rb, tb, outv,
            sem0, sem1, sem2, sem3):
        sems = [sem0, sem1, sem2, sem3]
        wid = lax.axis_index("s") * NC + lax.axis_index("c")
        base = wid * bpw

        pltpu.sync_copy(head_hbm.at[wid], hidx)
        pltpu.sync_copy(rel_hbm.at[wid], ridx)
        pltpu.sync_copy(tail_hbm.at[wid], tidx)

        copies = []
        for j in range(nchunks):
            rsl = pl.ds(j * CHUNK, CHUNK)
            sem = sems[j]
            copies.append(pltpu.async_copy(
                ent_hbm.at[hidx.at[j]], hrow.at[rsl], sem))
            copies.append(pltpu.async_copy(
                relemb_hbm.at[ridx.at[j]], rrow.at[rsl], sem))
            copies.append(pltpu.async_copy(
                ent_hbm.at[tidx.at[j]], trow.at[rsl], sem))
            copies.append(pltpu.async_copy(
                eb_hbm.at[hidx.at[j]], hb.at[rsl], sem))
            copies.append(pltpu.async_copy(
                rb_hbm.at[ridx.at[j]], rb.at[rsl], sem))
            copies.append(pltpu.async_copy(
                eb_hbm.at[tidx.at[j]], tb.at[rsl], sem))
        def group_body(g, carry):
            rows = g * L + lax.iota(jnp.int32, L)
            gsl = pl.ds(g * L, L)
            bias = hb[gsl] + rb[gsl] + tb[gsl]
            l1 = jnp.zeros((L,), jnp.float32)
            for d in range(DIM):
                dv = jnp.full((L,), d, jnp.int32)
                hv = plsc.load_gather(hrow, [rows, dv])
                rv = plsc.load_gather(rrow, [rows, dv])
                tv = plsc.load_gather(trow, [rows, dv])
                l1 = l1 + jnp.abs(hv + rv - tv)
            outv[gsl] = MARGIN - l1 + bias
            return carry

        gpc = CHUNK // L
        for j in range(nchunks):
            for c in copies[j * 6:(j + 1) * 6]:
                c.wait()
            lax.fori_loop(j * gpc, (j + 1) * gpc, group_body, 0)

        pltpu.sync_copy(outv, out_hbm.at[pl.ds(base, bpw)])

    return kge(head_r, rel_r, tail_r, entity_emb, relation_emb, e_bias,
               r_bias)


def kernel(head, relation, tail, entity_emb, relation_emb, e_bias, r_bias):
    batch = head.shape[0]
    bpw = batch // NW
    nchunks = bpw // CHUNK
    head_r = head.astype(jnp.int32).reshape(NW, nchunks, CHUNK)
    rel_r = relation.astype(jnp.int32).reshape(NW, nchunks, CHUNK)
    tail_r = tail.astype(jnp.int32).reshape(NW, nchunks, CHUNK)
    return _kge_sc(head_r, rel_r, tail_r, entity_emb, relation_emb,
                   e_bias.reshape(-1), r_bias.reshape(-1), batch)


# speedup vs baseline: 1.0027x; 1.0027x over previous
import functools

import jax
import jax.numpy as jnp
from jax import lax
from jax.experimental import pallas as pl
from jax.experimental.pallas import tpu as pltpu
from jax.experimental.pallas import tpu_sc as plsc

DIM = 64
MARGIN = 9.0
NC = 2
NS = 16
L = 16
NW = NC * NS
CHUNK = 128


def _kge_sc(head_r, rel_r, tail_r, entity_emb, relation_emb, e_bias, r_bias,
            batch):
    bpw = batch // NW
    nchunks = bpw // CHUNK

    mesh = plsc.VectorSubcoreMesh(core_axis_name="c", subcore_axis_name="s")

    @functools.partial(
        pl.kernel,
        mesh=mesh,
        out_type=jax.ShapeDtypeStruct((batch,), jnp.float32),
        compiler_params=pltpu.CompilerParams(
            use_tc_tiling_on_sc=False,
            needs_layout_passes=False,
        ),
        scratch_types=[
            pltpu.VMEM((nchunks, CHUNK), jnp.int32),
            pltpu.VMEM((nchunks, CHUNK), jnp.int32),
            pltpu.VMEM((nchunks, CHUNK), jnp.int32),
            pltpu.VMEM((bpw, DIM), jnp.float32),
            pltpu.VMEM((bpw, DIM), jnp.float32),
            pltpu.VMEM((bpw, DIM), jnp.float32),
            pltpu.VMEM((bpw,), jnp.float32),
            pltpu.VMEM((bpw,), jnp.float32),
            pltpu.VMEM((bpw,), jnp.float32),
            pltpu.VMEM((bpw,), jnp.float32),
            pltpu.SemaphoreType.DMA,
        ],
    )
    def kge(head_hbm, rel_hbm, tail_hbm, ent_hbm, relemb_hbm, eb_hbm, rb_hbm,
            out_hbm, hidx, ridx, tidx, hrow, rrow, trow, hb, rb, tb, outv,
            sem):
        wid = lax.axis_index("s") * NC + lax.axis_index("c")
        base = wid * bpw

        pltpu.sync_copy(head_hbm.at[wid], hidx)
        pltpu.sync_copy(rel_hbm.at[wid], ridx)
        pltpu.sync_copy(tail_hbm.at[wid], tidx)

        copies = []
        for j in range(nchunks):
            rsl = pl.ds(j * CHUNK, CHUNK)
            copies.append(pltpu.async_copy(
                ent_hbm.at[hidx.at[j]], hrow.at[rsl], sem))
            copies.append(pltpu.async_copy(
                relemb_hbm.at[ridx.at[j]], rrow.at[rsl], sem))
            copies.append(pltpu.async_copy(
                ent_hbm.at[tidx.at[j]], trow.at[rsl], sem))
            copies.append(pltpu.async_copy(
                eb_hbm.at[hidx.at[j]], hb.at[rsl], sem))
            copies.append(pltpu.async_copy(
                rb_hbm.at[ridx.at[j]], rb.at[rsl], sem))
            copies.append(pltpu.async_copy(
                eb_hbm.at[tidx.at[j]], tb.at[rsl], sem))
        for c in copies:
            c.wait()

        def group_body(g, carry):
            rows = g * L + lax.iota(jnp.int32, L)
            gsl = pl.ds(g * L, L)
            bias = hb[gsl] + rb[gsl] + tb[gsl]
            l1 = jnp.zeros((L,), jnp.float32)
            for d in range(DIM):
                dv = jnp.full((L,), d, jnp.int32)
                hv = plsc.load_gather(hrow, [rows, dv])
                rv = plsc.load_gather(rrow, [rows, dv])
                tv = plsc.load_gather(trow, [rows, dv])
                l1 = l1 + jnp.abs(hv + rv - tv)
            outv[gsl] = MARGIN - l1 + bias
            return carry

        lax.fori_loop(0, bpw // L, group_body, 0)
        pltpu.sync_copy(outv, out_hbm.at[pl.ds(base, bpw)])

    return kge(head_r, rel_r, tail_r, entity_emb, relation_emb, e_bias,
               r_bias)


def kernel(head, relation, tail, entity_emb, relation_emb, e_bias, r_bias):
    batch = head.shape[0]
    bpw = batch // NW
    nchunks = bpw // CHUNK
    head_r = head.astype(jnp.int32).reshape(NW, nchunks, CHUNK)
    rel_r = relation.astype(jnp.int32).reshape(NW, nchunks, CHUNK)
    tail_r = tail.astype(jnp.int32).reshape(NW, nchunks, CHUNK)
    return _kge_sc(head_r, rel_r, tail_r, entity_emb, relation_emb,
                   e_bias.reshape(-1), r_bias.reshape(-1), batch)
